# Initial kernel scaffold; baseline (speedup 1.0000x reference)
import jax
import jax.numpy as jnp
from jax.experimental import pallas as pl


def kernel(x, window_w, window_b, window_gamma, offset_w, offset_b, offset_gamma, kernel_w, kernel_b, kernel_gamma, v_w, v_b, out_w):
    raise NotImplementedError("write your pallas kernel here")


# speedup vs baseline: 4.6009x; 4.6009x over previous
import functools
import math

import jax
import jax.numpy as jnp
from jax.experimental import pallas as pl
from jax.experimental.pallas import tpu as pltpu


def _proj_kernel(x_ref, w_ref, bias_ref, wg_ref, og_ref, kg_ref,
                 v_ref, g_ref, c0_ref, *, H, K, max_window, min_window,
                 half_window_max, max_offset):
    xb = x_ref[...]
    y = jnp.dot(xb, w_ref[...], preferred_element_type=jnp.float32)
    y = y + bias_ref[...]
    HK = H * K
    wl = y[:, 0:H]
    ol = y[:, H:2 * H]
    kl = y[:, 2 * H:2 * H + HK]
    v = y[:, 2 * H + HK:]

    eps = 1e-6

    def rms(z, gamma):
        r = jnp.sqrt(jnp.mean(z * z, axis=-1, keepdims=True))
        return z / (r + eps) * gamma

    wn = rms(wl, wg_ref[...])
    window_sizes = min_window + jax.nn.sigmoid(wn) * (max_window - min_window)
    half_win = window_sizes * 0.5

    on = rms(ol, og_ref[...])
    c_off = jnp.tanh(on) * max_offset

    kn = rms(kl, kg_ref[...])
    kw = kn * jax.nn.sigmoid(kn)

    c0f = jnp.floor(c_off)
    frac = c_off - c0f
    c0_ref[...] = c0f.astype(jnp.int32)
    v_ref[...] = v

    wms = []
    for k in range(K):
        rel = float(k - K // 2)
        hard = 1.0 if abs(rel) <= half_window_max else 0.0
        wm = kw[:, k * H:(k + 1) * H] * jax.nn.sigmoid(half_win - abs(rel)) * hard
        wms.append(wm)

    one_m_frac = 1.0 - frac
    for j in range(K + 1):
        g = jnp.zeros_like(frac)
        if j < K:
            g = g + wms[j] * one_m_frac
        if j >= 1:
            g = g + wms[j - 1] * frac
        g_ref[:, j * H:(j + 1) * H] = g


def _conv_kernel(v_ref, g_ref, c0_ref, outw_ref, out_ref, *,
                 H, K, T, S, L, halo):
    i = pl.program_id(1)
    t0 = i * T
    s0 = pl.multiple_of(jnp.clip(t0 - halo, 0, L - S), 8)
    slab = v_ref[0, pl.ds(s0, S), :]
    C = slab.shape[-1]
    D = C // H

    iota_t = jax.lax.broadcasted_iota(jnp.int32, (T, 1), 0)
    iota_s = jax.lax.broadcasted_iota(jnp.int32, (T, S), 1)
    off = t0 - s0 - (K // 2)

    parts = []
    for h in range(H):
        base = iota_t + (c0_ref[:, h:h + 1] + off)
        band = jnp.zeros((T, S), dtype=jnp.float32)
        for j in range(K + 1):
            gv = g_ref[:, j * H + h:j * H + h + 1]
            eq = iota_s == (base + j)
            band = band + jnp.where(eq, gv, 0.0)
        hid = jax.lax.dot_general(
            band, slab[:, h * D:(h + 1) * D],
            (((1,), (0,)), ((), ())), preferred_element_type=jnp.float32)
        parts.append(hid)
    hidden = jnp.concatenate(parts, axis=1)
    o = jax.lax.dot_general(hidden, outw_ref[...],
                            (((1,), (1,)), ((), ())),
                            preferred_element_type=jnp.float32)
    out_ref[...] = o * jax.nn.sigmoid(o)


@jax.jit
def kernel(x, window_w, window_b, window_gamma, offset_w, offset_b,
           offset_gamma, kernel_w, kernel_b, kernel_gamma, v_w, v_b, out_w):
    B, L, C = x.shape
    H = window_w.shape[0]
    HK = kernel_w.shape[0]
    K = HK // H
    M = B * L
    max_window = min(int(math.sqrt(L)), K)
    half_window_max = max_window // 2
    max_offset = int(math.sqrt(L))
    min_window = 4.0

    kw_r = kernel_w.reshape(H, K, C).transpose(1, 0, 2).reshape(HK, C)
    kb_r = kernel_b.reshape(H, K).T.reshape(HK)
    kg_r = kernel_gamma.reshape(H, K).T.reshape(HK)

    w_cat = jnp.concatenate([window_w, offset_w, kw_r, v_w], axis=0).T
    b_cat = jnp.concatenate([window_b, offset_b, kb_r, v_b])[None, :]
    N = 2 * H + HK + C

    TM = 512
    xf = x.reshape(M, C)
    nproj = functools.partial(
        _proj_kernel, H=H, K=K, max_window=float(max_window),
        min_window=min_window, half_window_max=float(half_window_max),
        max_offset=float(max_offset))
    v, g, c0 = pl.pallas_call(
        nproj,
        grid=(M // TM,),
        in_specs=[
            pl.BlockSpec((TM, C), lambda m: (m, 0)),
            pl.BlockSpec((C, N), lambda m: (0, 0)),
            pl.BlockSpec((1, N), lambda m: (0, 0)),
            pl.BlockSpec((1, H), lambda m: (0, 0)),
            pl.BlockSpec((1, H), lambda m: (0, 0)),
            pl.BlockSpec((1, HK), lambda m: (0, 0)),
        ],
        out_specs=[
            pl.BlockSpec((TM, C), lambda m: (m, 0)),
            pl.BlockSpec((TM, (K + 1) * H), lambda m: (m, 0)),
            pl.BlockSpec((TM, H), lambda m: (m, 0)),
        ],
        out_shape=[
            jax.ShapeDtypeStruct((M, C), jnp.float32),
            jax.ShapeDtypeStruct((M, (K + 1) * H), jnp.float32),
            jax.ShapeDtypeStruct((M, H), jnp.int32),
        ],
    )(xf, w_cat, b_cat, window_gamma[None, :], offset_gamma[None, :],
      kg_r[None, :])

    T = 256
    halo = max_offset + K // 2
    S = T + 2 * halo
    NT = L // T
    nconv = functools.partial(_conv_kernel, H=H, K=K, T=T, S=S, L=L, halo=halo)
    out = pl.pallas_call(
        nconv,
        grid=(B, NT),
        in_specs=[
            pl.BlockSpec((1, L, C), lambda b, i: (b, 0, 0)),
            pl.BlockSpec((T, (K + 1) * H), lambda b, i, NT=NT: (b * NT + i, 0)),
            pl.BlockSpec((T, H), lambda b, i, NT=NT: (b * NT + i, 0)),
            pl.BlockSpec((C, C), lambda b, i: (0, 0)),
        ],
        out_specs=pl.BlockSpec((T, C), lambda b, i, NT=NT: (b * NT + i, 0)),
        out_shape=jax.ShapeDtypeStruct((M, C), jnp.float32),
    )(v.reshape(B, L, C), g, c0, out_w)
    return out.reshape(B, L, C)


# speedup vs baseline: 5.2180x; 1.1341x over previous
import functools
import math

import jax
import jax.numpy as jnp
from jax.experimental import pallas as pl
from jax.experimental.pallas import tpu as pltpu


def _proj_kernel(x_ref, w_ref, vw_ref, bias_ref, vb_ref, wg_ref, og_ref,
                 kg_ref, v_ref, g_ref, c0_ref, *, H, K, max_window,
                 min_window, half_window_max, max_offset):
    xb = x_ref[...]
    y = jnp.dot(xb, w_ref[...], preferred_element_type=jnp.float32)
    y = y + bias_ref[...]
    v = jnp.dot(xb.astype(jnp.bfloat16), vw_ref[...],
                preferred_element_type=jnp.float32) + vb_ref[...]
    HK = H * K
    wl = y[:, 0:H]
    ol = y[:, H:2 * H]
    kl = y[:, 2 * H:]

    eps = 1e-6

    def rms(z, gamma):
        r = jnp.sqrt(jnp.mean(z * z, axis=-1, keepdims=True))
        return z / (r + eps) * gamma

    wn = rms(wl, wg_ref[...])
    window_sizes = min_window + jax.nn.sigmoid(wn) * (max_window - min_window)
    half_win = window_sizes * 0.5

    on = rms(ol, og_ref[...])
    c_off = jnp.tanh(on) * max_offset

    kn = rms(kl, kg_ref[...])
    kw = kn * jax.nn.sigmoid(kn)

    c0f = jnp.floor(c_off)
    frac = c_off - c0f
    c0_ref[...] = c0f.astype(jnp.int32)
    v_ref[...] = v.astype(jnp.bfloat16)

    wms = []
    for k in range(K):
        rel = float(k - K // 2)
        hard = 1.0 if abs(rel) <= half_window_max else 0.0
        wm = kw[:, k * H:(k + 1) * H] * jax.nn.sigmoid(half_win - abs(rel)) * hard
        wms.append(wm)

    one_m_frac = 1.0 - frac
    for j in range(K + 1):
        g = jnp.zeros_like(frac)
        if j < K:
            g = g + wms[j] * one_m_frac
        if j >= 1:
            g = g + wms[j - 1] * frac
        g_ref[:, j * H:(j + 1) * H] = g.astype(jnp.bfloat16)


def _conv_kernel(v_ref, g_ref, c0_ref, outw_ref, out_ref, *,
                 H, K, T, S, L, halo):
    i = pl.program_id(1)
    t0 = i * T
    s0 = pl.multiple_of(jnp.clip(t0 - halo, 0, L - S), 8)
    slab = v_ref[0, pl.ds(s0, S), :]
    C = slab.shape[-1]
    D = C // H

    iota_t = jax.lax.broadcasted_iota(jnp.int32, (T, 1), 0)
    iota_s = jax.lax.broadcasted_iota(jnp.int32, (T, S), 1)
    off = t0 - s0 - (K // 2)

    parts = []
    for h in range(H):
        base = iota_t + (c0_ref[:, h:h + 1] + off)
        band = jnp.zeros((T, S), dtype=jnp.float32)
        for j in range(K + 1):
            gv = g_ref[:, j * H + h:j * H + h + 1].astype(jnp.float32)
            eq = iota_s == (base + j)
            band = band + jnp.where(eq, gv, 0.0)
        hid = jax.lax.dot_general(
            band.astype(jnp.bfloat16), slab[:, h * D:(h + 1) * D],
            (((1,), (0,)), ((), ())), preferred_element_type=jnp.float32)
        parts.append(hid.astype(jnp.bfloat16))
    hidden = jnp.concatenate(parts, axis=1)
    o = jax.lax.dot_general(hidden, outw_ref[...],
                            (((1,), (1,)), ((), ())),
                            preferred_element_type=jnp.float32)
    out_ref[...] = o * jax.nn.sigmoid(o)


@jax.jit
def kernel(x, window_w, window_b, window_gamma, offset_w, offset_b,
           offset_gamma, kernel_w, kernel_b, kernel_gamma, v_w, v_b, out_w):
    B, L, C = x.shape
    H = window_w.shape[0]
    HK = kernel_w.shape[0]
    K = HK // H
    M = B * L
    max_window = min(int(math.sqrt(L)), K)
    half_window_max = max_window // 2
    max_offset = int(math.sqrt(L))
    min_window = 4.0

    kw_r = kernel_w.reshape(H, K, C).transpose(1, 0, 2).reshape(HK, C)
    kb_r = kernel_b.reshape(H, K).T.reshape(HK)
    kg_r = kernel_gamma.reshape(H, K).T.reshape(HK)

    w_cat = jnp.concatenate([window_w, offset_w, kw_r], axis=0).T
    b_cat = jnp.concatenate([window_b, offset_b, kb_r])[None, :]
    N = 2 * H + HK
    vw_t = v_w.T.astype(jnp.bfloat16)

    TM = 512
    xf = x.reshape(M, C)
    nproj = functools.partial(
        _proj_kernel, H=H, K=K, max_window=float(max_window),
        min_window=min_window, half_window_max=float(half_window_max),
        max_offset=float(max_offset))
    v, g, c0 = pl.pallas_call(
        nproj,
        grid=(M // TM,),
        in_specs=[
            pl.BlockSpec((TM, C), lambda m: (m, 0)),
            pl.BlockSpec((C, N), lambda m: (0, 0)),
            pl.BlockSpec((C, C), lambda m: (0, 0)),
            pl.BlockSpec((1, N), lambda m: (0, 0)),
            pl.BlockSpec((1, C), lambda m: (0, 0)),
            pl.BlockSpec((1, H), lambda m: (0, 0)),
            pl.BlockSpec((1, H), lambda m: (0, 0)),
            pl.BlockSpec((1, HK), lambda m: (0, 0)),
        ],
        out_specs=[
            pl.BlockSpec((TM, C), lambda m: (m, 0)),
            pl.BlockSpec((TM, (K + 1) * H), lambda m: (m, 0)),
            pl.BlockSpec((TM, H), lambda m: (m, 0)),
        ],
        out_shape=[
            jax.ShapeDtypeStruct((M, C), jnp.bfloat16),
            jax.ShapeDtypeStruct((M, (K + 1) * H), jnp.bfloat16),
            jax.ShapeDtypeStruct((M, H), jnp.int32),
        ],
    )(xf, w_cat, vw_t, b_cat, v_b[None, :], window_gamma[None, :],
      offset_gamma[None, :], kg_r[None, :])

    T = 256
    halo = max_offset + K // 2
    S = T + 2 * halo
    NT = L // T
    nconv = functools.partial(_conv_kernel, H=H, K=K, T=T, S=S, L=L, halo=halo)
    out = pl.pallas_call(
        nconv,
        grid=(B, NT),
        in_specs=[
            pl.BlockSpec((1, L, C), lambda b, i: (b, 0, 0)),
            pl.BlockSpec((T, (K + 1) * H), lambda b, i, NT=NT: (b * NT + i, 0)),
            pl.BlockSpec((T, H), lambda b, i, NT=NT: (b * NT + i, 0)),
            pl.BlockSpec((C, C), lambda b, i: (0, 0)),
        ],
        out_specs=pl.BlockSpec((T, C), lambda b, i, NT=NT: (b * NT + i, 0)),
        out_shape=jax.ShapeDtypeStruct((M, C), jnp.float32),
    )(v.reshape(B, L, C), g, c0, out_w.astype(jnp.bfloat16))
    return out.reshape(B, L, C)


# speedup vs baseline: 13.5625x; 2.5992x over previous
import functools
import math

import jax
import jax.numpy as jnp
from jax.experimental import pallas as pl
from jax.experimental.pallas import tpu as pltpu


def _proj_kernel(x_ref, w_ref, vw_ref, bias_ref, vb_ref, wg_ref, og_ref,
                 kg_ref, v_ref, g_ref, c0_ref, *, H, K, max_window,
                 min_window, half_window_max, max_offset):
    xt = x_ref[...]
    y = jnp.dot(w_ref[...], xt, preferred_element_type=jnp.float32)
    y = y + bias_ref[...]
    v = jnp.dot(vw_ref[...], xt.astype(jnp.bfloat16),
                preferred_element_type=jnp.float32) + vb_ref[...]
    v_ref[...] = v.astype(jnp.bfloat16)

    HK = H * K
    wl = y[0:H, :]
    ol = y[H:2 * H, :]
    kl = y[2 * H:, :]

    eps = 1e-6

    def rms(z, gamma):
        r = jnp.sqrt(jnp.mean(z * z, axis=0, keepdims=True))
        return z / (r + eps) * gamma

    wn = rms(wl, wg_ref[...])
    window_sizes = min_window + jax.nn.sigmoid(wn) * (max_window - min_window)
    half_win = window_sizes * 0.5

    on = rms(ol, og_ref[...])
    c_off = jnp.tanh(on) * max_offset

    kn = rms(kl, kg_ref[...])
    kw = kn * jax.nn.sigmoid(kn)

    c0f = jnp.floor(c_off)
    frac = c_off - c0f
    c0_ref[...] = c0f.astype(jnp.int32)

    wms = []
    for k in range(K):
        rel = float(k - K // 2)
        hard = 1.0 if abs(rel) <= half_window_max else 0.0
        wm = kw[k * H:(k + 1) * H, :] * jax.nn.sigmoid(half_win - abs(rel)) * hard
        wms.append(wm)

    one_m_frac = 1.0 - frac
    for j in range(K + 1):
        g = jnp.zeros_like(frac)
        if j < K:
            g = g + wms[j] * one_m_frac
        if j >= 1:
            g = g + wms[j - 1] * frac
        g_ref[j * H:(j + 1) * H, :] = g.astype(jnp.bfloat16)


def _conv_kernel(v_ref, g_ref, c0_ref, outw_ref, out_ref, *,
                 H, K, T, S, L, halo):
    i = pl.program_id(1)
    t0 = i * T
    s0 = pl.multiple_of(jnp.clip(t0 - halo, 0, L - S), 128)
    C = v_ref.shape[0]
    D = C // H
    NCH = S // 128

    iota_t = jax.lax.broadcasted_iota(jnp.int32, (1, T), 1)
    iota_sub = jax.lax.broadcasted_iota(jnp.int32, (128, 1), 0).astype(jnp.int16)

    parts = []
    for h in range(H):
        base = iota_t + (c0_ref[h:h + 1, :] + (t0 - s0 - (K // 2)))
        base16 = base.astype(jnp.int16)
        gvs = [g_ref[j * H + h:j * H + h + 1, :] for j in range(K + 1)]
        hid = jnp.zeros((D, T), dtype=jnp.float32)
        for ci in range(NCH):
            d = iota_sub + jnp.int16(ci * 128) - base16
            band = jnp.zeros((128, T), dtype=jnp.bfloat16)
            for j in range(K + 1):
                band = jnp.where(d == jnp.int16(j), gvs[j], band)
            piece = v_ref[h * D:(h + 1) * D, pl.ds(s0 + ci * 128, 128)]
            hid = hid + jax.lax.dot_general(
                piece, band, (((1,), (0,)), ((), ())),
                preferred_element_type=jnp.float32)
        parts.append(hid.astype(jnp.bfloat16))
    hidden = jnp.concatenate(parts, axis=0)
    o = jnp.dot(outw_ref[...], hidden, preferred_element_type=jnp.float32)
    out_ref[...] = o * jax.nn.sigmoid(o)


@jax.jit
def kernel(x, window_w, window_b, window_gamma, offset_w, offset_b,
           offset_gamma, kernel_w, kernel_b, kernel_gamma, v_w, v_b, out_w):
    B, L, C = x.shape
    H = window_w.shape[0]
    HK = kernel_w.shape[0]
    K = HK // H
    M = B * L
    max_window = min(int(math.sqrt(L)), K)
    half_window_max = max_window // 2
    max_offset = int(math.sqrt(L))
    min_window = 4.0

    kw_r = kernel_w.reshape(H, K, C).transpose(1, 0, 2).reshape(HK, C)
    kb_r = kernel_b.reshape(H, K).T.reshape(HK)
    kg_r = kernel_gamma.reshape(H, K).T.reshape(HK)

    w_all = jnp.concatenate([window_w, offset_w, kw_r], axis=0)
    b_all = jnp.concatenate([window_b, offset_b, kb_r])[:, None]
    N = 2 * H + HK
    vw_bf = v_w.astype(jnp.bfloat16)

    xt = x.reshape(M, C).T

    TM = 512
    nproj = functools.partial(
        _proj_kernel, H=H, K=K, max_window=float(max_window),
        min_window=min_window, half_window_max=float(half_window_max),
        max_offset=float(max_offset))
    v, g, c0 = pl.pallas_call(
        nproj,
        grid=(M // TM,),
        in_specs=[
            pl.BlockSpec((C, TM), lambda m: (0, m)),
            pl.BlockSpec((N, C), lambda m: (0, 0)),
            pl.BlockSpec((C, C), lambda m: (0, 0)),
            pl.BlockSpec((N, 1), lambda m: (0, 0)),
            pl.BlockSpec((C, 1), lambda m: (0, 0)),
            pl.BlockSpec((H, 1), lambda m: (0, 0)),
            pl.BlockSpec((H, 1), lambda m: (0, 0)),
            pl.BlockSpec((HK, 1), lambda m: (0, 0)),
        ],
        out_specs=[
            pl.BlockSpec((C, TM), lambda m: (0, m)),
            pl.BlockSpec(((K + 1) * H, TM), lambda m: (0, m)),
            pl.BlockSpec((H, TM), lambda m: (0, m)),
        ],
        out_shape=[
            jax.ShapeDtypeStruct((C, M), jnp.bfloat16),
            jax.ShapeDtypeStruct(((K + 1) * H, M), jnp.bfloat16),
            jax.ShapeDtypeStruct((H, M), jnp.int32),
        ],
    )(xt, w_all, vw_bf, b_all, v_b[:, None], window_gamma[:, None],
      offset_gamma[:, None], kg_r[:, None])

    T = 256
    halo = 128
    S = T + 2 * halo
    NT = L // T
    nconv = functools.partial(_conv_kernel, H=H, K=K, T=T, S=S, L=L, halo=halo)
    out_t = pl.pallas_call(
        nconv,
        grid=(B, NT),
        in_specs=[
            pl.BlockSpec((C, L), lambda b, i: (0, b)),
            pl.BlockSpec(((K + 1) * H, T), lambda b, i, NT=NT: (0, b * NT + i)),
            pl.BlockSpec((H, T), lambda b, i, NT=NT: (0, b * NT + i)),
            pl.BlockSpec((C, C), lambda b, i: (0, 0)),
        ],
        out_specs=pl.BlockSpec((C, T), lambda b, i, NT=NT: (0, b * NT + i)),
        out_shape=jax.ShapeDtypeStruct((C, M), jnp.float32),
    )(v, g, c0, out_w.astype(jnp.bfloat16))
    return out_t.T.reshape(B, L, C)


# speedup vs baseline: 16.2360x; 1.1971x over previous
import functools
import math

import jax
import jax.numpy as jnp
from jax.experimental import pallas as pl
from jax.experimental.pallas import tpu as pltpu


def _proj_kernel(x_ref, w_ref, vw_ref, bias_ref, vb_ref, wg_ref, og_ref,
                 kg_ref, v_ref, g_ref, c0_ref, *, H, K, max_window,
                 min_window, half_window_max, max_offset):
    xb = x_ref[...]
    y = jax.lax.dot_general(w_ref[...], xb, (((1,), (1,)), ((), ())),
                            preferred_element_type=jnp.float32)
    y = y + bias_ref[...]
    v = jax.lax.dot_general(vw_ref[...], xb.astype(jnp.bfloat16),
                            (((1,), (1,)), ((), ())),
                            preferred_element_type=jnp.float32) + vb_ref[...]
    v_ref[...] = v.astype(jnp.bfloat16)

    HK = H * K
    wl = y[0:H, :]
    ol = y[H:2 * H, :]
    kl = y[2 * H:, :]

    eps = 1e-6

    def rms(z, gamma):
        r = jnp.sqrt(jnp.mean(z * z, axis=0, keepdims=True))
        return z / (r + eps) * gamma

    wn = rms(wl, wg_ref[...])
    window_sizes = min_window + jax.nn.sigmoid(wn) * (max_window - min_window)
    half_win = window_sizes * 0.5

    on = rms(ol, og_ref[...])
    c_off = jnp.tanh(on) * max_offset

    kn = rms(kl, kg_ref[...])
    kw = kn * jax.nn.sigmoid(kn)

    c0f = jnp.floor(c_off)
    frac = c_off - c0f
    c0_ref[...] = c0f.astype(jnp.int32)

    wms = []
    for k in range(K):
        rel = float(k - K // 2)
        hard = 1.0 if abs(rel) <= half_window_max else 0.0
        wm = kw[k * H:(k + 1) * H, :] * jax.nn.sigmoid(half_win - abs(rel)) * hard
        wms.append(wm)

    one_m_frac = 1.0 - frac
    for j in range(K + 1):
        g = jnp.zeros_like(frac)
        if j < K:
            g = g + wms[j] * one_m_frac
        if j >= 1:
            g = g + wms[j - 1] * frac
        g_ref[j * H:(j + 1) * H, :] = g.astype(jnp.bfloat16)


def _conv_kernel(v_ref, g_ref, c0_ref, outw_ref, out_ref, *,
                 H, K, T, S, L, halo):
    i = pl.program_id(1)
    t0 = i * T
    s0 = pl.multiple_of(jnp.clip(t0 - halo, 0, L - S), 128)
    C = v_ref.shape[0]
    D = C // H
    NCH = S // 128

    iota_t = jax.lax.broadcasted_iota(jnp.int32, (1, T), 1)
    iota_sub = jax.lax.broadcasted_iota(jnp.int32, (128, 1), 0).astype(jnp.int16)

    parts = []
    for h in range(H):
        base = iota_t + (c0_ref[h:h + 1, :] + (t0 - s0 - (K // 2)))
        base16 = base.astype(jnp.int16)
        gvs = [g_ref[j * H + h:j * H + h + 1, :] for j in range(K + 1)]
        hid = jnp.zeros((D, T), dtype=jnp.float32)
        for ci in range(NCH):
            d = iota_sub + jnp.int16(ci * 128) - base16
            band = jnp.zeros((128, T), dtype=jnp.bfloat16)
            for j in range(K + 1):
                band = jnp.where(d == jnp.int16(j), gvs[j], band)
            piece = v_ref[h * D:(h + 1) * D, pl.ds(s0 + ci * 128, 128)]
            hid = hid + jax.lax.dot_general(
                piece, band, (((1,), (0,)), ((), ())),
                preferred_element_type=jnp.float32)
        parts.append(hid.astype(jnp.bfloat16))
    hidden = jnp.concatenate(parts, axis=0)
    o = jax.lax.dot_general(hidden, outw_ref[...],
                            (((0,), (1,)), ((), ())),
                            preferred_element_type=jnp.float32)
    out_ref[...] = o * jax.nn.sigmoid(o)


@jax.jit
def kernel(x, window_w, window_b, window_gamma, offset_w, offset_b,
           offset_gamma, kernel_w, kernel_b, kernel_gamma, v_w, v_b, out_w):
    B, L, C = x.shape
    H = window_w.shape[0]
    HK = kernel_w.shape[0]
    K = HK // H
    M = B * L
    max_window = min(int(math.sqrt(L)), K)
    half_window_max = max_window // 2
    max_offset = int(math.sqrt(L))
    min_window = 4.0

    kw_r = kernel_w.reshape(H, K, C).transpose(1, 0, 2).reshape(HK, C)
    kb_r = kernel_b.reshape(H, K).T.reshape(HK)
    kg_r = kernel_gamma.reshape(H, K).T.reshape(HK)

    w_all = jnp.concatenate([window_w, offset_w, kw_r], axis=0)
    b_all = jnp.concatenate([window_b, offset_b, kb_r])[:, None]
    N = 2 * H + HK
    vw_bf = v_w.astype(jnp.bfloat16)

    xf = x.reshape(M, C)

    TM = 512
    nproj = functools.partial(
        _proj_kernel, H=H, K=K, max_window=float(max_window),
        min_window=min_window, half_window_max=float(half_window_max),
        max_offset=float(max_offset))
    v, g, c0 = pl.pallas_call(
        nproj,
        grid=(M // TM,),
        in_specs=[
            pl.BlockSpec((TM, C), lambda m: (m, 0)),
            pl.BlockSpec((N, C), lambda m: (0, 0)),
            pl.BlockSpec((C, C), lambda m: (0, 0)),
            pl.BlockSpec((N, 1), lambda m: (0, 0)),
            pl.BlockSpec((C, 1), lambda m: (0, 0)),
            pl.BlockSpec((H, 1), lambda m: (0, 0)),
            pl.BlockSpec((H, 1), lambda m: (0, 0)),
            pl.BlockSpec((HK, 1), lambda m: (0, 0)),
        ],
        out_specs=[
            pl.BlockSpec((C, TM), lambda m: (0, m)),
            pl.BlockSpec(((K + 1) * H, TM), lambda m: (0, m)),
            pl.BlockSpec((H, TM), lambda m: (0, m)),
        ],
        out_shape=[
            jax.ShapeDtypeStruct((C, M), jnp.bfloat16),
            jax.ShapeDtypeStruct(((K + 1) * H, M), jnp.bfloat16),
            jax.ShapeDtypeStruct((H, M), jnp.int32),
        ],
    )(xf, w_all, vw_bf, b_all, v_b[:, None], window_gamma[:, None],
      offset_gamma[:, None], kg_r[:, None])

    T = 256
    halo = 128
    S = T + 2 * halo
    NT = L // T
    nconv = functools.partial(_conv_kernel, H=H, K=K, T=T, S=S, L=L, halo=halo)
    out_t = pl.pallas_call(
        nconv,
        grid=(B, NT),
        in_specs=[
            pl.BlockSpec((C, L), lambda b, i: (0, b)),
            pl.BlockSpec(((K + 1) * H, T), lambda b, i, NT=NT: (0, b * NT + i)),
            pl.BlockSpec((H, T), lambda b, i, NT=NT: (0, b * NT + i)),
            pl.BlockSpec((C, C), lambda b, i: (0, 0)),
        ],
        out_specs=pl.BlockSpec((T, C), lambda b, i, NT=NT: (b * NT + i, 0)),
        out_shape=jax.ShapeDtypeStruct((M, C), jnp.float32),
    )(v, g, c0, out_w.astype(jnp.bfloat16))
    return out_t.reshape(B, L, C)


# speedup vs baseline: 16.6446x; 1.0252x over previous
import functools
import math

import jax
import jax.numpy as jnp
from jax.experimental import pallas as pl
from jax.experimental.pallas import tpu as pltpu


def _proj_kernel(x_ref, w_ref, vw_ref, bias_ref, vb_ref, wg_ref, og_ref,
                 kg_ref, v_ref, g_ref, c0_ref, *, H, K, max_window,
                 min_window, half_window_max, max_offset):
    xb = x_ref[...]
    y = jax.lax.dot_general(w_ref[...], xb, (((1,), (1,)), ((), ())),
                            preferred_element_type=jnp.float32)
    y = y + bias_ref[...]
    v = jax.lax.dot_general(vw_ref[...], xb.astype(jnp.bfloat16),
                            (((1,), (1,)), ((), ())),
                            preferred_element_type=jnp.float32) + vb_ref[...]
    v_ref[...] = v.astype(jnp.bfloat16)

    HK = H * K
    wl = y[0:H, :]
    ol = y[H:2 * H, :]
    kl = y[2 * H:, :]

    eps = 1e-6

    def rms(z, gamma):
        r = jnp.sqrt(jnp.mean(z * z, axis=0, keepdims=True))
        return z / (r + eps) * gamma

    wn = rms(wl, wg_ref[...])
    window_sizes = min_window + jax.nn.sigmoid(wn) * (max_window - min_window)
    half_win = window_sizes * 0.5

    on = rms(ol, og_ref[...])
    c_off = jnp.tanh(on) * max_offset

    kn = rms(kl, kg_ref[...])
    kw = kn * jax.nn.sigmoid(kn)

    c0f = jnp.floor(c_off)
    frac = c_off - c0f
    c0_ref[...] = c0f.astype(jnp.int32)

    wms = []
    for k in range(K):
        rel = float(k - K // 2)
        hard = 1.0 if abs(rel) <= half_window_max else 0.0
        wm = kw[k * H:(k + 1) * H, :] * jax.nn.sigmoid(half_win - abs(rel)) * hard
        wms.append(wm)

    one_m_frac = 1.0 - frac
    for j in range(K + 1):
        g = jnp.zeros_like(frac)
        if j < K:
            g = g + wms[j] * one_m_frac
        if j >= 1:
            g = g + wms[j - 1] * frac
        g_ref[j * H:(j + 1) * H, :] = g.astype(jnp.bfloat16)


def _conv_kernel(v_ref, g_ref, c0_ref, outw_ref, out_ref, *,
                 H, K, T, S, L, halo):
    i = pl.program_id(1)
    t0 = i * T
    s0 = pl.multiple_of(jnp.clip(t0 - halo, 0, L - S), 128)
    C = v_ref.shape[0]
    D = C // H
    NCH = S // 128

    iota_t = jax.lax.broadcasted_iota(jnp.int32, (1, T), 1)
    iota_sub = jax.lax.broadcasted_iota(jnp.int32, (128, 1), 0).astype(jnp.int16)

    parts = []
    for h in range(H):
        base = iota_t + (c0_ref[h:h + 1, :] + (t0 - s0 - (K // 2)))
        base16 = base.astype(jnp.int16)
        gvs = [g_ref[j * H + h:j * H + h + 1, :] for j in range(K + 1)]
        hid = jnp.zeros((D, T), dtype=jnp.float32)
        for ci in range(NCH):
            d = iota_sub + jnp.int16(ci * 128) - base16
            band = jnp.zeros((128, T), dtype=jnp.bfloat16)
            for j in range(K + 1):
                band = jnp.where(d == jnp.int16(j), gvs[j], band)
            piece = v_ref[h * D:(h + 1) * D, pl.ds(s0 + ci * 128, 128)]
            hid = hid + jax.lax.dot_general(
                piece, band, (((1,), (0,)), ((), ())),
                preferred_element_type=jnp.float32)
        parts.append(hid.astype(jnp.bfloat16))
    hidden = jnp.concatenate(parts, axis=0)
    o = jax.lax.dot_general(hidden, outw_ref[...],
                            (((0,), (1,)), ((), ())),
                            preferred_element_type=jnp.float32)
    out_ref[...] = o * jax.nn.sigmoid(o)


@jax.jit
def kernel(x, window_w, window_b, window_gamma, offset_w, offset_b,
           offset_gamma, kernel_w, kernel_b, kernel_gamma, v_w, v_b, out_w):
    B, L, C = x.shape
    H = window_w.shape[0]
    HK = kernel_w.shape[0]
    K = HK // H
    M = B * L
    max_window = min(int(math.sqrt(L)), K)
    half_window_max = max_window // 2
    max_offset = int(math.sqrt(L))
    min_window = 4.0

    kw_r = kernel_w.reshape(H, K, C).transpose(1, 0, 2).reshape(HK, C)
    kb_r = kernel_b.reshape(H, K).T.reshape(HK)
    kg_r = kernel_gamma.reshape(H, K).T.reshape(HK)

    w_all = jnp.concatenate([window_w, offset_w, kw_r], axis=0)
    b_all = jnp.concatenate([window_b, offset_b, kb_r])[:, None]
    N = 2 * H + HK
    vw_bf = v_w.astype(jnp.bfloat16)

    xf = x.reshape(M, C)

    TM = 512
    nproj = functools.partial(
        _proj_kernel, H=H, K=K, max_window=float(max_window),
        min_window=min_window, half_window_max=float(half_window_max),
        max_offset=float(max_offset))
    v, g, c0 = pl.pallas_call(
        nproj,
        grid=(M // TM,),
        in_specs=[
            pl.BlockSpec((TM, C), lambda m: (m, 0)),
            pl.BlockSpec((N, C), lambda m: (0, 0)),
            pl.BlockSpec((C, C), lambda m: (0, 0)),
            pl.BlockSpec((N, 1), lambda m: (0, 0)),
            pl.BlockSpec((C, 1), lambda m: (0, 0)),
            pl.BlockSpec((H, 1), lambda m: (0, 0)),
            pl.BlockSpec((H, 1), lambda m: (0, 0)),
            pl.BlockSpec((HK, 1), lambda m: (0, 0)),
        ],
        out_specs=[
            pl.BlockSpec((C, TM), lambda m: (0, m)),
            pl.BlockSpec(((K + 1) * H, TM), lambda m: (0, m)),
            pl.BlockSpec((H, TM), lambda m: (0, m)),
        ],
        out_shape=[
            jax.ShapeDtypeStruct((C, M), jnp.bfloat16),
            jax.ShapeDtypeStruct(((K + 1) * H, M), jnp.bfloat16),
            jax.ShapeDtypeStruct((H, M), jnp.int32),
        ],
    )(xf, w_all, vw_bf, b_all, v_b[:, None], window_gamma[:, None],
      offset_gamma[:, None], kg_r[:, None])

    T = 128
    halo = 128
    S = T + 2 * halo
    NT = L // T
    nconv = functools.partial(_conv_kernel, H=H, K=K, T=T, S=S, L=L, halo=halo)
    out_t = pl.pallas_call(
        nconv,
        grid=(B, NT),
        in_specs=[
            pl.BlockSpec((C, L), lambda b, i: (0, b)),
            pl.BlockSpec(((K + 1) * H, T), lambda b, i, NT=NT: (0, b * NT + i)),
            pl.BlockSpec((H, T), lambda b, i, NT=NT: (0, b * NT + i)),
            pl.BlockSpec((C, C), lambda b, i: (0, 0)),
        ],
        out_specs=pl.BlockSpec((T, C), lambda b, i, NT=NT: (b * NT + i, 0)),
        out_shape=jax.ShapeDtypeStruct((M, C), jnp.float32),
    )(v, g, c0, out_w.astype(jnp.bfloat16))
    return out_t.reshape(B, L, C)


# speedup vs baseline: 16.9701x; 1.0196x over previous
import functools
import math

import jax
import jax.numpy as jnp
from jax.experimental import pallas as pl
from jax.experimental.pallas import tpu as pltpu


def _fused_kernel(x_ref, w_ref, vw_ref, bias_ref, vb_ref, wg_ref, og_ref,
                  kg_ref, outw_ref, out_ref, v_s, g_s, c0_s, *,
                  H, K, max_window, min_window, half_window_max, max_offset,
                  TM, NP, T, S, L, NT, halo):
    s = pl.program_id(0)

    @pl.when(s < NP)
    def _proj():
        m0 = pl.multiple_of(s * TM, TM)
        xb = x_ref[...]
        y = jax.lax.dot_general(w_ref[...], xb, (((1,), (1,)), ((), ())),
                                preferred_element_type=jnp.float32)
        y = y + bias_ref[...]
        v = jax.lax.dot_general(vw_ref[...], xb.astype(jnp.bfloat16),
                                (((1,), (1,)), ((), ())),
                                preferred_element_type=jnp.float32)
        v = v + vb_ref[...]
        v_s[:, pl.ds(m0, TM)] = v.astype(jnp.bfloat16)

        HK = H * K
        wl = y[0:H, :]
        ol = y[H:2 * H, :]
        kl = y[2 * H:, :]

        eps = 1e-6

        def rms(z, gamma):
            r = jnp.sqrt(jnp.mean(z * z, axis=0, keepdims=True))
            return z / (r + eps) * gamma

        wn = rms(wl, wg_ref[...])
        window_sizes = (min_window
                        + jax.nn.sigmoid(wn) * (max_window - min_window))
        half_win = window_sizes * 0.5

        on = rms(ol, og_ref[...])
        c_off = jnp.tanh(on) * max_offset

        kn = rms(kl, kg_ref[...])
        kw = kn * jax.nn.sigmoid(kn)

        c0f = jnp.floor(c_off)
        frac = c_off - c0f
        c0_s[:, pl.ds(m0, TM)] = c0f.astype(jnp.int32)

        wms = []
        for k in range(K):
            rel = float(k - K // 2)
            hard = 1.0 if abs(rel) <= half_window_max else 0.0
            wm = (kw[k * H:(k + 1) * H, :]
                  * jax.nn.sigmoid(half_win - abs(rel)) * hard)
            wms.append(wm)

        one_m_frac = 1.0 - frac
        for j in range(K + 1):
            g = jnp.zeros_like(frac)
            if j < K:
                g = g + wms[j] * one_m_frac
            if j >= 1:
                g = g + wms[j - 1] * frac
            g_s[j * H:(j + 1) * H, pl.ds(m0, TM)] = g.astype(jnp.bfloat16)

    @pl.when(s >= NP)
    def _conv():
        ii = s - NP
        b = ii // NT
        i = ii % NT
        t0 = i * T
        m0 = pl.multiple_of(b * L + t0, T)
        s0 = pl.multiple_of(
            b * L + jnp.clip(t0 - halo, 0, L - S), 128)
        C = v_s.shape[0]
        D = C // H
        NCH = S // 128

        iota_t = jax.lax.broadcasted_iota(jnp.int32, (1, T), 1)
        iota_sub = jax.lax.broadcasted_iota(
            jnp.int32, (128, 1), 0).astype(jnp.int16)

        parts = []
        for h in range(H):
            base = iota_t + (c0_s[h:h + 1, pl.ds(m0, T)]
                             + (m0 - s0 - (K // 2)))
            base16 = base.astype(jnp.int16)
            gvs = [g_s[j * H + h:j * H + h + 1, pl.ds(m0, T)]
                   for j in range(K + 1)]
            hid = jnp.zeros((D, T), dtype=jnp.float32)
            for ci in range(NCH):
                d = iota_sub + jnp.int16(ci * 128) - base16
                band = jnp.zeros((128, T), dtype=jnp.bfloat16)
                for j in range(K + 1):
                    band = jnp.where(d == jnp.int16(j), gvs[j], band)
                piece = v_s[h * D:(h + 1) * D, pl.ds(s0 + ci * 128, 128)]
                hid = hid + jax.lax.dot_general(
                    piece, band, (((1,), (0,)), ((), ())),
                    preferred_element_type=jnp.float32)
            parts.append(hid.astype(jnp.bfloat16))
        hidden = jnp.concatenate(parts, axis=0)
        o = jax.lax.dot_general(hidden, outw_ref[...],
                                (((0,), (1,)), ((), ())),
                                preferred_element_type=jnp.float32)
        out_ref[...] = o * jax.nn.sigmoid(o)


@jax.jit
def kernel(x, window_w, window_b, window_gamma, offset_w, offset_b,
           offset_gamma, kernel_w, kernel_b, kernel_gamma, v_w, v_b, out_w):
    B, L, C = x.shape
    H = window_w.shape[0]
    HK = kernel_w.shape[0]
    K = HK // H
    M = B * L
    max_window = min(int(math.sqrt(L)), K)
    half_window_max = max_window // 2
    max_offset = int(math.sqrt(L))
    min_window = 4.0

    kw_r = kernel_w.reshape(H, K, C).transpose(1, 0, 2).reshape(HK, C)
    kb_r = kernel_b.reshape(H, K).T.reshape(HK)
    kg_r = kernel_gamma.reshape(H, K).T.reshape(HK)

    w_all = jnp.concatenate([window_w, offset_w, kw_r], axis=0)
    b_all = jnp.concatenate([window_b, offset_b, kb_r])[:, None]
    N = 2 * H + HK
    vw_bf = v_w.astype(jnp.bfloat16)

    xf = x.reshape(M, C)

    TM = 512
    NP = M // TM
    T = 128
    halo = 128
    S = T + 2 * halo
    NT = L // T

    body = functools.partial(
        _fused_kernel, H=H, K=K, max_window=float(max_window),
        min_window=min_window, half_window_max=float(half_window_max),
        max_offset=float(max_offset), TM=TM, NP=NP, T=T, S=S, L=L, NT=NT,
        halo=halo)
    out = pl.pallas_call(
        body,
        grid=(NP + B * NT,),
        in_specs=[
            pl.BlockSpec((TM, C), lambda s, NP=NP: (jnp.minimum(s, NP - 1), 0)),
            pl.BlockSpec((N, C), lambda s: (0, 0)),
            pl.BlockSpec((C, C), lambda s: (0, 0)),
            pl.BlockSpec((N, 1), lambda s: (0, 0)),
            pl.BlockSpec((C, 1), lambda s: (0, 0)),
            pl.BlockSpec((H, 1), lambda s: (0, 0)),
            pl.BlockSpec((H, 1), lambda s: (0, 0)),
            pl.BlockSpec((HK, 1), lambda s: (0, 0)),
            pl.BlockSpec((C, C), lambda s: (0, 0)),
        ],
        out_specs=pl.BlockSpec(
            (T, C), lambda s, NP=NP: (jnp.maximum(s - NP, 0), 0)),
        out_shape=jax.ShapeDtypeStruct((M, C), jnp.float32),
        scratch_shapes=[
            pltpu.VMEM((C, M), jnp.bfloat16),
            pltpu.VMEM(((K + 1) * H, M), jnp.bfloat16),
            pltpu.VMEM((H, M), jnp.int32),
        ],
    )(xf, w_all, vw_bf, b_all, v_b[:, None], window_gamma[:, None],
      offset_gamma[:, None], kg_r[:, None], out_w.astype(jnp.bfloat16))
    return out.reshape(B, L, C)
